# Initial kernel scaffold; baseline (speedup 1.0000x reference)
import jax
import jax.numpy as jnp
from jax.experimental import pallas as pl


def kernel(h, coords, batch, params):
    raise NotImplementedError("write your pallas kernel here")


# speedup vs baseline: 157.7759x; 157.7759x over previous
import functools

import jax
import jax.numpy as jnp
from jax.experimental import pallas as pl

_B = 32
_HID = 32
_TILE = 512


def _up(t, B, N):
    avg = jnp.stack([t[:, :-1], t[:, 1:]], axis=2).mean(axis=2)
    tmp = jnp.stack([t[:, :-1], avg], axis=2).reshape(B, 2 * (N - 1), t.shape[-1])
    return jnp.concatenate([t[:, :1], tmp, jnp.tile(t[:, -1:], (1, 3, 1))], axis=1)


def _ln(x, w, b):
    mu = jnp.mean(x, axis=-1, keepdims=True)
    var = jnp.mean((x - mu) * (x - mu), axis=-1, keepdims=True)
    return (x - mu) / jnp.sqrt(var + 1e-5) * w + b


def _silu(x):
    return x * jax.nn.sigmoid(x)


def _cross(a, b):
    a0, a1, a2 = a[:, 0:1], a[:, 1:2], a[:, 2:3]
    b0, b1, b2 = b[:, 0:1], b[:, 1:2], b[:, 2:3]
    return jnp.concatenate(
        [a1 * b2 - a2 * b1, a2 * b0 - a0 * b2, a0 * b1 - a1 * b0], axis=1)


def _dot(a, b):
    return jax.lax.dot_general(a, b, (((1,), (0,)), ((), ())),
                               preferred_element_type=jnp.float32)


def _egnn_pool_kernel(hp0, hp1, hp2, x0, x1, x2,
                      ei_w, ei_b, em_w1a, em_w1b, em_b1, em_w2, em_b2,
                      em_w3, em_b3, lne_w, lne_b,
                      ee_w1h, ee_w1c, ee_w1d, ee_w1e, ee_b1, ee_w2, ee_b2,
                      ec_w1, ec_b1, ec_w2, ex_w1, ex_b1, ex_w2,
                      en_w1a, en_w1b, en_b1, en_w2, en_b2,
                      eo_w, eo_b, lnh_w, lnh_b,
                      h_out, x_out):
    hps = (hp0[...], hp1[...], hp2[...])
    xs = (x0[...], x1[...], x2[...])
    hp = (hps[0] + hps[1] + hps[2]) * (1.0 / 3.0)
    xp = (xs[0] + xs[1] + xs[2]) * (1.0 / 3.0)
    hr = _dot(hp, ei_w[...]) + ei_b[...]

    agg = jnp.zeros_like(hr)
    xacc = xp
    for k in range(3):
        hc_raw = hps[k]
        xc = xs[k]
        ea = jnp.maximum(_dot(hp, em_w1a[...]) + _dot(hc_raw, em_w1b[...])
                         + em_b1[...], 0.0)
        ea = jnp.maximum(_dot(ea, em_w2[...]) + em_b2[...], 0.0)
        ea = _dot(ea, em_w3[...]) + em_b3[...]
        ea = _ln(ea, lne_w[...], lne_b[...])
        hc = _dot(hc_raw, ei_w[...]) + ei_b[...]
        diff = xp - xc
        d2 = jnp.sum(diff * diff, axis=-1, keepdims=True)
        dn = diff / (jnp.sqrt(d2 + 1e-8) + 1.0)
        m = _silu(_dot(hr, ee_w1h[...]) + _dot(hc, ee_w1c[...])
                  + d2 * ee_w1d[...] + _dot(ea, ee_w1e[...]) + ee_b1[...])
        m = _silu(_dot(m, ee_w2[...]) + ee_b2[...])
        tcoef = _dot(_silu(_dot(m, ec_w1[...]) + ec_b1[...]), ec_w2[...])
        xcoef = _dot(_silu(_dot(m, ex_w1[...]) + ex_b1[...]), ex_w2[...])
        cr = _cross(xp, xc)
        crn = jnp.sqrt(jnp.sum(cr * cr, axis=-1, keepdims=True))
        cr = cr / (crn + 1.0)
        xacc = xacc + dn * tcoef + cr * xcoef
        agg = agg + m

    h2 = hr + _dot(_silu(_dot(hr, en_w1a[...]) + _dot(agg, en_w1b[...])
                         + en_b1[...]), en_w2[...]) + en_b2[...]
    h2 = _dot(h2, eo_w[...]) + eo_b[...]
    h_out[...] = _ln(h2, lnh_w[...], lnh_b[...])
    x_out[...] = xacc


@functools.partial(jax.jit, static_argnames=())
def _run(h, coords, p):
    B, C = _B, _HID
    N = h.shape[0] // B
    out_size = (N - 1) * 2 + 2
    R = B * out_size

    h_up = _up(h.reshape(B, N, C), B, N)
    x_up = _up(coords.reshape(B, N, 3), B, N)
    hps = [h_up[:, k:k + out_size].reshape(R, C) for k in range(3)]
    xs = [x_up[:, k:k + out_size].reshape(R, 3) for k in range(3)]

    def v(name):
        return p[name].reshape(1, -1)

    weights = [
        p['ei_w'], v('ei_b'),
        p['em_w1'][:C], p['em_w1'][C:], v('em_b1'),
        p['em_w2'], v('em_b2'), p['em_w3'], v('em_b3'),
        v('lne_w'), v('lne_b'),
        p['ee_w1'][:C], p['ee_w1'][C:2 * C], p['ee_w1'][2 * C:2 * C + 1],
        p['ee_w1'][2 * C + 1:], v('ee_b1'),
        p['ee_w2'], v('ee_b2'),
        p['ec_w1'], v('ec_b1'), p['ec_w2'],
        p['ex_w1'], v('ex_b1'), p['ex_w2'],
        p['en_w1'][:C], p['en_w1'][C:], v('en_b1'),
        p['en_w2'], v('en_b2'),
        p['eo_w'], v('eo_b'),
        v('lnh_w'), v('lnh_b'),
    ]

    grid = (R // _TILE,)
    row_spec_h = pl.BlockSpec((_TILE, C), lambda i: (i, 0))
    row_spec_x = pl.BlockSpec((_TILE, 3), lambda i: (i, 0))
    w_specs = [pl.BlockSpec(w.shape, lambda i: (0, 0)) for w in weights]

    h_out, x_out = pl.pallas_call(
        _egnn_pool_kernel,
        grid=grid,
        in_specs=[row_spec_h] * 3 + [row_spec_x] * 3 + w_specs,
        out_specs=[row_spec_h, row_spec_x],
        out_shape=[
            jax.ShapeDtypeStruct((R, C), jnp.float32),
            jax.ShapeDtypeStruct((R, 3), jnp.float32),
        ],
    )(*hps, *xs, *weights)
    return h_out, x_out


def kernel(h, coords, batch, params):
    del batch
    return _run(h, coords, params)


# speedup vs baseline: 239.7277x; 1.5194x over previous
import functools

import jax
import jax.numpy as jnp
from jax.experimental import pallas as pl

_B = 32
_HID = 32
_G = 8


def _ln(x, w, b):
    mu = jnp.mean(x, axis=-1, keepdims=True)
    var = jnp.mean((x - mu) * (x - mu), axis=-1, keepdims=True)
    return (x - mu) / jnp.sqrt(var + 1e-5) * w + b


def _silu(x):
    return x * jax.nn.sigmoid(x)


def _cross(a, b):
    a0, a1, a2 = a[:, 0:1], a[:, 1:2], a[:, 2:3]
    b0, b1, b2 = b[:, 0:1], b[:, 1:2], b[:, 2:3]
    return jnp.concatenate(
        [a1 * b2 - a2 * b1, a2 * b0 - a0 * b2, a0 * b1 - a1 * b0], axis=1)


def _dot(a, b):
    return jax.lax.dot_general(a, b, (((1,), (0,)), ((), ())),
                               preferred_element_type=jnp.float32)


def _shift_pairs(t):
    e = jnp.concatenate(
        [t[:, :1], (t[:, :-1] + t[:, 1:]) * 0.5, t[:, -1:]], axis=1)
    o = jnp.concatenate([t, t[:, -1:]], axis=1)
    return e, o


def _egnn_pool_kernel(t_ref, c_ref,
                      em_w1, em_b1, em_w2, em_b2, em_w3, em_b3,
                      lne_w, lne_b, ei_w, ei_b,
                      ee_w1, ee_b1, ee_w2, ee_b2,
                      ec_w1, ec_b1, ec_w2, ex_w1, ex_b1, ex_w2,
                      en_w1, en_b1, en_w2, en_b2,
                      eo_w, eo_b, lnh_w, lnh_b,
                      h_out, x_out):
    C = _HID
    G = t_ref.shape[0]
    R = G * 128
    eh, oh = _shift_pairs(t_ref[...])
    ec_, oc = _shift_pairs(c_ref[...])

    def cols(e, o, d):
        c0 = jnp.concatenate([e[:, :64], o[:, :64]], axis=1).reshape(R, d)
        c1 = jnp.concatenate([o[:, :64], e[:, 1:65]], axis=1).reshape(R, d)
        c2 = jnp.concatenate([e[:, 1:65], o[:, 1:65]], axis=1).reshape(R, d)
        return c0, c1, c2

    hcols = cols(eh, oh, C)
    xcols = cols(ec_, oc, 3)
    hp = (hcols[0] + hcols[1] + hcols[2]) * (1.0 / 3.0)
    xp = (xcols[0] + xcols[1] + xcols[2]) * (1.0 / 3.0)

    em_w1a, em_w1b = em_w1[0:C], em_w1[C:2 * C]
    ee_w1h, ee_w1c = ee_w1[0:C], ee_w1[C:2 * C]
    ee_w1d, ee_w1e = ee_w1[2 * C:2 * C + 1], ee_w1[2 * C + 1:3 * C + 1]
    en_w1a, en_w1b = en_w1[0:C], en_w1[C:2 * C]

    hr = _dot(hp, ei_w[...]) + ei_b[...]
    hp_em = _dot(hp, em_w1a)
    hr_ee = _dot(hr, ee_w1h)

    agg = jnp.zeros_like(hr)
    xacc = xp
    for k in range(3):
        hc_raw = hcols[k]
        xc = xcols[k]
        ea = jnp.maximum(hp_em + _dot(hc_raw, em_w1b) + em_b1[...], 0.0)
        ea = jnp.maximum(_dot(ea, em_w2[...]) + em_b2[...], 0.0)
        ea = _dot(ea, em_w3[...]) + em_b3[...]
        ea = _ln(ea, lne_w[...], lne_b[...])
        hc = _dot(hc_raw, ei_w[...]) + ei_b[...]
        diff = xp - xc
        d2 = jnp.sum(diff * diff, axis=-1, keepdims=True)
        dn = diff / (jnp.sqrt(d2 + 1e-8) + 1.0)
        m = _silu(hr_ee + _dot(hc, ee_w1c) + d2 * ee_w1d + _dot(ea, ee_w1e)
                  + ee_b1[...])
        m = _silu(_dot(m, ee_w2[...]) + ee_b2[...])
        tcoef = _dot(_silu(_dot(m, ec_w1[...]) + ec_b1[...]), ec_w2[...])
        xcoef = _dot(_silu(_dot(m, ex_w1[...]) + ex_b1[...]), ex_w2[...])
        cr = _cross(xp, xc)
        crn = jnp.sqrt(jnp.sum(cr * cr, axis=-1, keepdims=True))
        cr = cr / (crn + 1.0)
        xacc = xacc + dn * tcoef + cr * xcoef
        agg = agg + m

    h2 = hr + _dot(_silu(_dot(hr, en_w1a) + _dot(agg, en_w1b)
                         + en_b1[...]), en_w2[...]) + en_b2[...]
    h2 = _dot(h2, eo_w[...]) + eo_b[...]
    h2 = _ln(h2, lnh_w[...], lnh_b[...])

    h2 = h2.reshape(G, 128, C)
    xacc = xacc.reshape(G, 128, 3)
    h_out[:, 0:128:2, :] = h2[:, :64]
    h_out[:, 1:128:2, :] = h2[:, 64:]
    x_out[:, 0:128:2, :] = xacc[:, :64]
    x_out[:, 1:128:2, :] = xacc[:, 64:]


@jax.jit
def _run(h, coords, p):
    B, C = _B, _HID
    N = h.shape[0] // B
    out_size = 2 * N
    t = h.reshape(B, N, C)
    c = coords.reshape(B, N, 3)

    def v(name):
        return p[name].reshape(1, -1)

    weights = [
        p['em_w1'], v('em_b1'), p['em_w2'], v('em_b2'), p['em_w3'], v('em_b3'),
        v('lne_w'), v('lne_b'), p['ei_w'], v('ei_b'),
        p['ee_w1'], v('ee_b1'), p['ee_w2'], v('ee_b2'),
        p['ec_w1'], v('ec_b1'), p['ec_w2'],
        p['ex_w1'], v('ex_b1'), p['ex_w2'],
        p['en_w1'], v('en_b1'), p['en_w2'], v('en_b2'),
        p['eo_w'], v('eo_b'), v('lnh_w'), v('lnh_b'),
    ]

    grid = (B // _G,)
    t_spec = pl.BlockSpec((_G, N, C), lambda i: (i, 0, 0))
    c_spec = pl.BlockSpec((_G, N, 3), lambda i: (i, 0, 0))
    w_specs = [pl.BlockSpec(w.shape, lambda i: (0,) * w.ndim) for w in weights]

    h_out, x_out = pl.pallas_call(
        _egnn_pool_kernel,
        grid=grid,
        in_specs=[t_spec, c_spec] + w_specs,
        out_specs=[
            pl.BlockSpec((_G, out_size, C), lambda i: (i, 0, 0)),
            pl.BlockSpec((_G, out_size, 3), lambda i: (i, 0, 0)),
        ],
        out_shape=[
            jax.ShapeDtypeStruct((B, out_size, C), jnp.float32),
            jax.ShapeDtypeStruct((B, out_size, 3), jnp.float32),
        ],
    )(t, c, *weights)
    return h_out.reshape(B * out_size, C), x_out.reshape(B * out_size, 3)


def kernel(h, coords, batch, params):
    del batch
    return _run(h, coords, params)


# speedup vs baseline: 254.0275x; 1.0597x over previous
import functools

import jax
import jax.numpy as jnp
from jax.experimental import pallas as pl

_B = 32
_HID = 32
_G = 8


def _ln_mm(x, w, b, jmat):
    mu = _dot(x, jmat)
    xc = x - mu
    var = _dot(xc * xc, jmat)
    return xc * jax.lax.rsqrt(var + 1e-5) * w + b


def _silu(x):
    return x / (1.0 + jnp.exp(-x))


def _cross(a, b):
    a0, a1, a2 = a[:, 0:1], a[:, 1:2], a[:, 2:3]
    b0, b1, b2 = b[:, 0:1], b[:, 1:2], b[:, 2:3]
    return jnp.concatenate(
        [a1 * b2 - a2 * b1, a2 * b0 - a0 * b2, a0 * b1 - a1 * b0], axis=1)


def _dot(a, b):
    return jax.lax.dot_general(a, b, (((1,), (0,)), ((), ())),
                               preferred_element_type=jnp.float32)


def _shift_pairs(t):
    e = jnp.concatenate(
        [t[:, :1], (t[:, :-1] + t[:, 1:]) * 0.5, t[:, -1:]], axis=1)
    o = jnp.concatenate([t, t[:, -1:]], axis=1)
    return e, o


def _egnn_pool_kernel(t_ref, c_ref,
                      em_w1, em_b1, em_w2, em_b2, em_w3, em_b3,
                      lne_w, lne_b, ei_w, ei_b,
                      ee_w1, ee_b1, ee_w2, ee_b2,
                      ec_w1, ec_b1, ec_w2, ex_w1, ex_b1, ex_w2,
                      en_w1, en_b1, en_w2, en_b2,
                      eo_w, eo_b, lnh_w, lnh_b,
                      h_out, x_out):
    C = _HID
    G = t_ref.shape[0]
    R = G * 128
    eh, oh = _shift_pairs(t_ref[...])
    ec_, oc = _shift_pairs(c_ref[...])

    def cols(e, o, d):
        c0 = jnp.concatenate([e[:, :64], o[:, :64]], axis=1).reshape(R, d)
        c1 = jnp.concatenate([o[:, :64], e[:, 1:65]], axis=1).reshape(R, d)
        c2 = jnp.concatenate([e[:, 1:65], o[:, 1:65]], axis=1).reshape(R, d)
        return c0, c1, c2

    hcols = cols(eh, oh, C)
    xcols = cols(ec_, oc, 3)
    hp = (hcols[0] + hcols[1] + hcols[2]) * (1.0 / 3.0)
    xp = (xcols[0] + xcols[1] + xcols[2]) * (1.0 / 3.0)

    em_w1a, em_w1b = em_w1[0:C], em_w1[C:2 * C]
    ee_w1h = ee_w1[0:C]
    ee_w1d, ee_w1e = ee_w1[2 * C:2 * C + 1], ee_w1[2 * C + 1:3 * C + 1]
    en_w1a, en_w1b = en_w1[0:C], en_w1[C:2 * C]

    jmat = jnp.full((C, C), 1.0 / C, jnp.float32)
    ones3c = jnp.full((3, C), 1.0, jnp.float32)
    ones33 = jnp.full((3, 3), 1.0, jnp.float32)
    ee_w1c = ee_w1[C:2 * C]
    wc_comp = _dot(ei_w[...], ee_w1c)
    bc_comp = _dot(ei_b[...], ee_w1c) + ee_b1[...]
    ec_w2_3 = jnp.concatenate([ec_w2[...]] * 3, axis=1)
    ex_w2_3 = jnp.concatenate([ex_w2[...]] * 3, axis=1)

    hr = _dot(hp, ei_w[...]) + ei_b[...]
    hp_em = _dot(hp, em_w1a)
    hr_ee = _dot(hr, ee_w1h)

    agg = jnp.zeros_like(hr)
    xacc = xp
    for k in range(3):
        hc_raw = hcols[k]
        xc = xcols[k]
        ea = jnp.maximum(hp_em + _dot(hc_raw, em_w1b) + em_b1[...], 0.0)
        ea = jnp.maximum(_dot(ea, em_w2[...]) + em_b2[...], 0.0)
        ea = _dot(ea, em_w3[...]) + em_b3[...]
        ea = _ln_mm(ea, lne_w[...], lne_b[...], jmat)
        diff = xp - xc
        dsq = diff * diff
        d2c = _dot(dsq, ones3c)
        d23 = _dot(dsq, ones33)
        dn = diff * (1.0 / (jnp.sqrt(d23 + 1e-8) + 1.0))
        m = _silu(hr_ee + _dot(hc_raw, wc_comp) + d2c * ee_w1d
                  + _dot(ea, ee_w1e) + bc_comp)
        m = _silu(_dot(m, ee_w2[...]) + ee_b2[...])
        tcoef = _dot(_silu(_dot(m, ec_w1[...]) + ec_b1[...]), ec_w2_3)
        xcoef = _dot(_silu(_dot(m, ex_w1[...]) + ex_b1[...]), ex_w2_3)
        cr = _cross(xp, xc)
        crn2 = _dot(cr * cr, ones33)
        cr = cr * (1.0 / (jnp.sqrt(crn2) + 1.0))
        xacc = xacc + dn * tcoef + cr * xcoef
        agg = agg + m

    h2 = hr + _dot(_silu(_dot(hr, en_w1a) + _dot(agg, en_w1b)
                         + en_b1[...]), en_w2[...]) + en_b2[...]
    h2 = _dot(h2, eo_w[...]) + eo_b[...]
    h2 = _ln_mm(h2, lnh_w[...], lnh_b[...], jmat)

    h2 = h2.reshape(G, 128, C)
    xacc = xacc.reshape(G, 128, 3)
    h_out[:, 0:128:2, :] = h2[:, :64]
    h_out[:, 1:128:2, :] = h2[:, 64:]
    x_out[:, 0:128:2, :] = xacc[:, :64]
    x_out[:, 1:128:2, :] = xacc[:, 64:]


@jax.jit
def _run(h, coords, p):
    B, C = _B, _HID
    N = h.shape[0] // B
    out_size = 2 * N
    t = h.reshape(B, N, C)
    c = coords.reshape(B, N, 3)

    def v(name):
        return p[name].reshape(1, -1)

    weights = [
        p['em_w1'], v('em_b1'), p['em_w2'], v('em_b2'), p['em_w3'], v('em_b3'),
        v('lne_w'), v('lne_b'), p['ei_w'], v('ei_b'),
        p['ee_w1'], v('ee_b1'), p['ee_w2'], v('ee_b2'),
        p['ec_w1'], v('ec_b1'), p['ec_w2'],
        p['ex_w1'], v('ex_b1'), p['ex_w2'],
        p['en_w1'], v('en_b1'), p['en_w2'], v('en_b2'),
        p['eo_w'], v('eo_b'), v('lnh_w'), v('lnh_b'),
    ]

    grid = (B // _G,)
    t_spec = pl.BlockSpec((_G, N, C), lambda i: (i, 0, 0))
    c_spec = pl.BlockSpec((_G, N, 3), lambda i: (i, 0, 0))
    w_specs = [pl.BlockSpec(w.shape, lambda i: (0,) * w.ndim) for w in weights]

    h_out, x_out = pl.pallas_call(
        _egnn_pool_kernel,
        grid=grid,
        in_specs=[t_spec, c_spec] + w_specs,
        out_specs=[
            pl.BlockSpec((_G, out_size, C), lambda i: (i, 0, 0)),
            pl.BlockSpec((_G, out_size, 3), lambda i: (i, 0, 0)),
        ],
        out_shape=[
            jax.ShapeDtypeStruct((B, out_size, C), jnp.float32),
            jax.ShapeDtypeStruct((B, out_size, 3), jnp.float32),
        ],
    )(t, c, *weights)
    return h_out.reshape(B * out_size, C), x_out.reshape(B * out_size, 3)


def kernel(h, coords, batch, params):
    del batch
    return _run(h, coords, params)


# speedup vs baseline: 261.4254x; 1.0291x over previous
import functools

import jax
import jax.numpy as jnp
from jax.experimental import pallas as pl

_B = 32
_HID = 32
_G = 16


def _ln_mm(x, w, b, jmat):
    mu = _dot(x, jmat)
    xc = x - mu
    var = _dot(xc * xc, jmat)
    return xc * jax.lax.rsqrt(var + 1e-5) * w + b


def _silu(x):
    return x / (1.0 + jnp.exp(-x))


def _cross(a, b):
    a0, a1, a2 = a[:, 0:1], a[:, 1:2], a[:, 2:3]
    b0, b1, b2 = b[:, 0:1], b[:, 1:2], b[:, 2:3]
    return jnp.concatenate(
        [a1 * b2 - a2 * b1, a2 * b0 - a0 * b2, a0 * b1 - a1 * b0], axis=1)


def _dot(a, b):
    return jax.lax.dot_general(a, b, (((1,), (0,)), ((), ())),
                               preferred_element_type=jnp.float32)


def _shift_pairs(t):
    e = jnp.concatenate(
        [t[:, :1], (t[:, :-1] + t[:, 1:]) * 0.5, t[:, -1:]], axis=1)
    o = jnp.concatenate([t, t[:, -1:]], axis=1)
    return e, o


def _egnn_pool_kernel(t_ref, c_ref,
                      em_w1, em_b1, em_w2, em_b2, em_w3, em_b3,
                      lne_w, lne_b, ei_w, ei_b,
                      ee_w1, ee_b1, ee_w2, ee_b2,
                      ec_w1, ec_b1, ec_w2, ex_w1, ex_b1, ex_w2,
                      en_w1, en_b1, en_w2, en_b2,
                      eo_w, eo_b, lnh_w, lnh_b,
                      h_out, x_out):
    C = _HID
    G = t_ref.shape[0]
    R = G * 128
    eh, oh = _shift_pairs(t_ref[...])
    ec_, oc = _shift_pairs(c_ref[...])

    def cols(e, o, d):
        c0 = jnp.concatenate([e[:, :64], o[:, :64]], axis=1).reshape(R, d)
        c1 = jnp.concatenate([o[:, :64], e[:, 1:65]], axis=1).reshape(R, d)
        c2 = jnp.concatenate([e[:, 1:65], o[:, 1:65]], axis=1).reshape(R, d)
        return c0, c1, c2

    hcols = cols(eh, oh, C)
    xcols = cols(ec_, oc, 3)
    hp = (hcols[0] + hcols[1] + hcols[2]) * (1.0 / 3.0)
    xp = (xcols[0] + xcols[1] + xcols[2]) * (1.0 / 3.0)

    em_w1a, em_w1b = em_w1[0:C], em_w1[C:2 * C]
    ee_w1h = ee_w1[0:C]
    ee_w1d, ee_w1e = ee_w1[2 * C:2 * C + 1], ee_w1[2 * C + 1:3 * C + 1]
    en_w1a, en_w1b = en_w1[0:C], en_w1[C:2 * C]

    jmat = jnp.full((C, C), 1.0 / C, jnp.float32)
    ones3c = jnp.full((3, C), 1.0, jnp.float32)
    ones33 = jnp.full((3, 3), 1.0, jnp.float32)
    ee_w1c = ee_w1[C:2 * C]
    wc_comp = _dot(ei_w[...], ee_w1c)
    bc_comp = _dot(ei_b[...], ee_w1c) + ee_b1[...]
    ec_w2_3 = jnp.concatenate([ec_w2[...]] * 3, axis=1)
    ex_w2_3 = jnp.concatenate([ex_w2[...]] * 3, axis=1)

    hr = _dot(hp, ei_w[...]) + ei_b[...]
    hp_em = _dot(hp, em_w1a)
    hr_ee = _dot(hr, ee_w1h)

    agg = jnp.zeros_like(hr)
    xacc = xp
    for k in range(3):
        hc_raw = hcols[k]
        xc = xcols[k]
        ea = jnp.maximum(hp_em + _dot(hc_raw, em_w1b) + em_b1[...], 0.0)
        ea = jnp.maximum(_dot(ea, em_w2[...]) + em_b2[...], 0.0)
        ea = _dot(ea, em_w3[...]) + em_b3[...]
        ea = _ln_mm(ea, lne_w[...], lne_b[...], jmat)
        diff = xp - xc
        dsq = diff * diff
        d2c = _dot(dsq, ones3c)
        d23 = _dot(dsq, ones33)
        dn = diff * (1.0 / (jnp.sqrt(d23 + 1e-8) + 1.0))
        m = _silu(hr_ee + _dot(hc_raw, wc_comp) + d2c * ee_w1d
                  + _dot(ea, ee_w1e) + bc_comp)
        m = _silu(_dot(m, ee_w2[...]) + ee_b2[...])
        tcoef = _dot(_silu(_dot(m, ec_w1[...]) + ec_b1[...]), ec_w2_3)
        xcoef = _dot(_silu(_dot(m, ex_w1[...]) + ex_b1[...]), ex_w2_3)
        cr = _cross(xp, xc)
        crn2 = _dot(cr * cr, ones33)
        cr = cr * (1.0 / (jnp.sqrt(crn2) + 1.0))
        xacc = xacc + dn * tcoef + cr * xcoef
        agg = agg + m

    h2 = hr + _dot(_silu(_dot(hr, en_w1a) + _dot(agg, en_w1b)
                         + en_b1[...]), en_w2[...]) + en_b2[...]
    h2 = _dot(h2, eo_w[...]) + eo_b[...]
    h2 = _ln_mm(h2, lnh_w[...], lnh_b[...], jmat)

    h2 = h2.reshape(G, 128, C)
    xacc = xacc.reshape(G, 128, 3)
    h_out[:, 0:128:2, :] = h2[:, :64]
    h_out[:, 1:128:2, :] = h2[:, 64:]
    x_out[:, 0:128:2, :] = xacc[:, :64]
    x_out[:, 1:128:2, :] = xacc[:, 64:]


@jax.jit
def _run(h, coords, p):
    B, C = _B, _HID
    N = h.shape[0] // B
    out_size = 2 * N
    t = h.reshape(B, N, C)
    c = coords.reshape(B, N, 3)

    def v(name):
        return p[name].reshape(1, -1)

    weights = [
        p['em_w1'], v('em_b1'), p['em_w2'], v('em_b2'), p['em_w3'], v('em_b3'),
        v('lne_w'), v('lne_b'), p['ei_w'], v('ei_b'),
        p['ee_w1'], v('ee_b1'), p['ee_w2'], v('ee_b2'),
        p['ec_w1'], v('ec_b1'), p['ec_w2'],
        p['ex_w1'], v('ex_b1'), p['ex_w2'],
        p['en_w1'], v('en_b1'), p['en_w2'], v('en_b2'),
        p['eo_w'], v('eo_b'), v('lnh_w'), v('lnh_b'),
    ]

    grid = (B // _G,)
    t_spec = pl.BlockSpec((_G, N, C), lambda i: (i, 0, 0))
    c_spec = pl.BlockSpec((_G, N, 3), lambda i: (i, 0, 0))
    w_specs = [pl.BlockSpec(w.shape, lambda i: (0,) * w.ndim) for w in weights]

    h_out, x_out = pl.pallas_call(
        _egnn_pool_kernel,
        grid=grid,
        in_specs=[t_spec, c_spec] + w_specs,
        out_specs=[
            pl.BlockSpec((_G, out_size, C), lambda i: (i, 0, 0)),
            pl.BlockSpec((_G, out_size, 3), lambda i: (i, 0, 0)),
        ],
        out_shape=[
            jax.ShapeDtypeStruct((B, out_size, C), jnp.float32),
            jax.ShapeDtypeStruct((B, out_size, 3), jnp.float32),
        ],
    )(t, c, *weights)
    return h_out.reshape(B * out_size, C), x_out.reshape(B * out_size, 3)


def kernel(h, coords, batch, params):
    del batch
    return _run(h, coords, params)
